# Initial kernel scaffold; baseline (speedup 1.0000x reference)
import jax
import jax.numpy as jnp
from jax.experimental import pallas as pl


def kernel(x, edge_index, batch, W1, b1, g1, be1, W2, b2, g2, be2, W3, b3, g3, be3, mW1, mb1, mW2, mb2, mW3, mb3, oW, ob):
    raise NotImplementedError("write your pallas kernel here")


# speedup vs baseline: 10.5617x; 10.5617x over previous
import functools

import jax
import jax.numpy as jnp
from jax import lax
from jax.experimental import pallas as pl
from jax.experimental.pallas import tpu as pltpu
from jax.experimental.pallas import tpu_sc as plsc

NC = 2
NS = 16
NW = NC * NS
KE = 80
G = 64
OUT_DIM = 2
LATENT = 32
BN_ROWS = 1000


def _sc_mesh():
    return plsc.VectorSubcoreMesh(core_axis_name="c", subcore_axis_name="s")


def _row_chunks(n):
    cps = ((n + NS - 1) // NS + 7) // 8 * 8
    last = n - (NS - 1) * cps
    assert 0 < last <= cps
    return cps, last


def _sc_degree(dst, n):
    e = dst.shape[0]
    epw = e // NW
    nch = epw // KE
    cps, last = _row_chunks(n)
    ones = jnp.ones((KE, 128), jnp.float32)
    zeros = jnp.zeros((cps, 128), jnp.float32)

    @functools.partial(
        pl.kernel,
        out_type=jax.ShapeDtypeStruct((NC, n, 128), jnp.float32),
        mesh=_sc_mesh(),
        scratch_types=[
            pltpu.VMEM((1, KE), jnp.int32),
            pltpu.VMEM((KE, 128), jnp.float32),
            pltpu.VMEM_SHARED((n, 128), jnp.float32),
        ],
    )
    def k(dst_hbm, ones_hbm, zeros_hbm, out_hbm, idx_v, ones_v, acc_sh):
        c = lax.axis_index("c")
        s = lax.axis_index("s")
        wid = c * NS + s

        @pl.when(s < NS - 1)
        def _():
            pltpu.sync_copy(zeros_hbm, acc_sh.at[pl.ds(s * cps, cps)])

        @pl.when(s == NS - 1)
        def _():
            pltpu.sync_copy(zeros_hbm.at[pl.ds(0, last)],
                            acc_sh.at[pl.ds((NS - 1) * cps, last)])

        pltpu.sync_copy(ones_hbm, ones_v)
        plsc.subcore_barrier()
        base = wid * epw

        @pl.loop(0, nch)
        def _(ci):
            pltpu.sync_copy(dst_hbm.at[pl.ds(base + ci * KE, KE)], idx_v.at[0])
            pltpu.sync_copy(ones_v, acc_sh.at[idx_v.at[0]], add=True)

        plsc.subcore_barrier()

        @pl.when(s < NS - 1)
        def _():
            pltpu.sync_copy(acc_sh.at[pl.ds(s * cps, cps)],
                            out_hbm.at[c, pl.ds(s * cps, cps)])

        @pl.when(s == NS - 1)
        def _():
            pltpu.sync_copy(acc_sh.at[pl.ds((NS - 1) * cps, last)],
                            out_hbm.at[c, pl.ds((NS - 1) * cps, last)])

    return k(dst, ones, zeros)


def _sc_message(hp, src, dst):
    n, h = hp.shape
    e = src.shape[0]
    epw = e // NW
    nch = epw // KE
    cps, last = _row_chunks(n)
    zeros = jnp.zeros((cps, h), jnp.float32)

    @functools.partial(
        pl.kernel,
        out_type=jax.ShapeDtypeStruct((NC, n, h), jnp.float32),
        mesh=_sc_mesh(),
        scratch_types=[
            pltpu.VMEM((1, KE), jnp.int32),
            pltpu.VMEM((1, KE), jnp.int32),
            pltpu.VMEM((KE, h), jnp.float32),
            pltpu.VMEM_SHARED((n, h), jnp.float32),
        ],
    )
    def k(hp_hbm, src_hbm, dst_hbm, zeros_hbm, out_hbm, si_v, di_v, rows_v, acc_sh):
        c = lax.axis_index("c")
        s = lax.axis_index("s")
        wid = c * NS + s

        @pl.when(s < NS - 1)
        def _():
            pltpu.sync_copy(zeros_hbm, acc_sh.at[pl.ds(s * cps, cps)])

        @pl.when(s == NS - 1)
        def _():
            pltpu.sync_copy(zeros_hbm.at[pl.ds(0, last)],
                            acc_sh.at[pl.ds((NS - 1) * cps, last)])

        plsc.subcore_barrier()
        base = wid * epw

        @pl.loop(0, nch)
        def _(ci):
            pltpu.sync_copy(src_hbm.at[pl.ds(base + ci * KE, KE)], si_v.at[0])
            pltpu.sync_copy(dst_hbm.at[pl.ds(base + ci * KE, KE)], di_v.at[0])
            pltpu.sync_copy(hp_hbm.at[si_v.at[0]], rows_v)
            pltpu.sync_copy(rows_v, acc_sh.at[di_v.at[0]], add=True)

        plsc.subcore_barrier()

        @pl.when(s < NS - 1)
        def _():
            pltpu.sync_copy(acc_sh.at[pl.ds(s * cps, cps)],
                            out_hbm.at[c, pl.ds(s * cps, cps)])

        @pl.when(s == NS - 1)
        def _():
            pltpu.sync_copy(acc_sh.at[pl.ds((NS - 1) * cps, last)],
                            out_hbm.at[c, pl.ds((NS - 1) * cps, last)])

    return k(hp, src, dst, zeros)


def _tc_mm1(x, W, degp):
    n, d = x.shape
    h = W.shape[1]
    nb = n // BN_ROWS

    def body(x_ref, w_ref, degp_ref, hp_ref, dinv_ref):
        dp = degp_ref[...]
        deg = dp[0, :, 0:1] + dp[1, :, 0:1] + 1.0
        dinv = lax.rsqrt(deg)
        hp_ref[...] = dinv * jnp.dot(x_ref[...], w_ref[...],
                                     preferred_element_type=jnp.float32)
        dinv_ref[...] = dinv

    return pl.pallas_call(
        body,
        grid=(nb,),
        in_specs=[
            pl.BlockSpec((BN_ROWS, d), lambda i: (i, 0)),
            pl.BlockSpec((d, h), lambda i: (0, 0)),
            pl.BlockSpec((NC, BN_ROWS, 128), lambda i: (0, i, 0)),
        ],
        out_specs=[
            pl.BlockSpec((BN_ROWS, h), lambda i: (i, 0)),
            pl.BlockSpec((BN_ROWS, 1), lambda i: (i, 0)),
        ],
        out_shape=[
            jax.ShapeDtypeStruct((n, h), jnp.float32),
            jax.ShapeDtypeStruct((n, 1), jnp.float32),
        ],
    )(x, W, degp)


def _tc_merge_stats(P, hp, dinv, b):
    n, h = hp.shape
    nb = n // BN_ROWS

    def body(p_ref, hp_ref, dinv_ref, b_ref, t_ref, st_ref):
        i = pl.program_id(0)
        p = p_ref[...]
        t = dinv_ref[...] * (p[0] + p[1] + hp_ref[...]) + b_ref[...]
        t_ref[...] = t

        @pl.when(i == 0)
        def _():
            st_ref[...] = jnp.zeros_like(st_ref)

        st_ref[...] += jnp.stack([jnp.sum(t, 0), jnp.sum(t * t, 0)])

    return pl.pallas_call(
        body,
        grid=(nb,),
        in_specs=[
            pl.BlockSpec((NC, BN_ROWS, h), lambda i: (0, i, 0)),
            pl.BlockSpec((BN_ROWS, h), lambda i: (i, 0)),
            pl.BlockSpec((BN_ROWS, 1), lambda i: (i, 0)),
            pl.BlockSpec((1, h), lambda i: (0, 0)),
        ],
        out_specs=[
            pl.BlockSpec((BN_ROWS, h), lambda i: (i, 0)),
            pl.BlockSpec((2, h), lambda i: (0, 0)),
        ],
        out_shape=[
            jax.ShapeDtypeStruct((n, h), jnp.float32),
            jax.ShapeDtypeStruct((2, h), jnp.float32),
        ],
    )(P, hp, dinv, b)


def _tc_bn_mm(t, st, g, be, W, dinv):
    n, h = t.shape
    h2 = W.shape[1]
    nb = n // BN_ROWS
    inv_n = 1.0 / n

    def body(t_ref, st_ref, g_ref, be_ref, w_ref, dinv_ref, o_ref):
        st_v = st_ref[...]
        mu = st_v[0:1] * inv_n
        var = st_v[1:2] * inv_n - mu * mu
        y = (t_ref[...] - mu) * lax.rsqrt(var + 1e-5) * g_ref[...] + be_ref[...]
        y = jnp.maximum(y, 0.0)
        o_ref[...] = dinv_ref[...] * jnp.dot(y, w_ref[...],
                                             preferred_element_type=jnp.float32)

    return pl.pallas_call(
        body,
        grid=(nb,),
        in_specs=[
            pl.BlockSpec((BN_ROWS, h), lambda i: (i, 0)),
            pl.BlockSpec((2, h), lambda i: (0, 0)),
            pl.BlockSpec((1, h), lambda i: (0, 0)),
            pl.BlockSpec((1, h), lambda i: (0, 0)),
            pl.BlockSpec((h, h2), lambda i: (0, 0)),
            pl.BlockSpec((BN_ROWS, 1), lambda i: (i, 0)),
        ],
        out_specs=pl.BlockSpec((BN_ROWS, h2), lambda i: (i, 0)),
        out_shape=jax.ShapeDtypeStruct((n, h2), jnp.float32),
    )(t, st, g, be, W, dinv)


def _tc_final(t, st, g, be, batch2d, mW1, mb1, mW2, mb2, mW3, mb3, oW, ob):
    n, h = t.shape
    nb = n // BN_ROWS
    inv_n = 1.0 / n
    h2 = mW2.shape[1]
    h3 = mW3.shape[1]
    od = oW.shape[1]

    def body(t_ref, st_ref, g_ref, be_ref, bat_ref,
             mw1_ref, mb1_ref, mw2_ref, mb2_ref, mw3_ref, mb3_ref,
             ow_ref, ob_ref, out_ref, pool_ref, cnt_ref):
        i = pl.program_id(0)
        st_v = st_ref[...]
        mu = st_v[0:1] * inv_n
        var = st_v[1:2] * inv_n - mu * mu
        y = (t_ref[...] - mu) * lax.rsqrt(var + 1e-5) * g_ref[...] + be_ref[...]
        y = jnp.maximum(y, 0.0)
        seg = bat_ref[0]
        gids = lax.broadcasted_iota(jnp.int32, (G, BN_ROWS), 0)
        onehot = jnp.where(seg == gids, 1.0, 0.0)

        @pl.when(i == 0)
        def _():
            pool_ref[...] = jnp.zeros_like(pool_ref)
            cnt_ref[...] = jnp.zeros_like(cnt_ref)

        pool_ref[...] += jnp.dot(onehot, y, preferred_element_type=jnp.float32)
        cnt_ref[...] += jnp.sum(onehot, axis=1, keepdims=True)

        @pl.when(i == nb - 1)
        def _():
            pooled = pool_ref[...] / jnp.maximum(cnt_ref[...], 1.0)
            z = jnp.maximum(jnp.dot(pooled, mw1_ref[...],
                                    preferred_element_type=jnp.float32) + mb1_ref[...], 0.0)
            z = jnp.maximum(jnp.dot(z, mw2_ref[...],
                                    preferred_element_type=jnp.float32) + mb2_ref[...], 0.0)
            z = jnp.maximum(jnp.dot(z, mw3_ref[...],
                                    preferred_element_type=jnp.float32) + mb3_ref[...], 0.0)
            out_ref[...] = jnp.dot(z, ow_ref[...],
                                   preferred_element_type=jnp.float32) + ob_ref[...]

    return pl.pallas_call(
        body,
        grid=(nb,),
        in_specs=[
            pl.BlockSpec((BN_ROWS, h), lambda i: (i, 0)),
            pl.BlockSpec((2, h), lambda i: (0, 0)),
            pl.BlockSpec((1, h), lambda i: (0, 0)),
            pl.BlockSpec((1, h), lambda i: (0, 0)),
            pl.BlockSpec((1, 1, BN_ROWS), lambda i: (i, 0, 0)),
            pl.BlockSpec((h, h), lambda i: (0, 0)),
            pl.BlockSpec((1, h), lambda i: (0, 0)),
            pl.BlockSpec((h, h2), lambda i: (0, 0)),
            pl.BlockSpec((1, h2), lambda i: (0, 0)),
            pl.BlockSpec((h2, h3), lambda i: (0, 0)),
            pl.BlockSpec((1, h3), lambda i: (0, 0)),
            pl.BlockSpec((h3, od), lambda i: (0, 0)),
            pl.BlockSpec((1, od), lambda i: (0, 0)),
        ],
        out_specs=pl.BlockSpec((G, od), lambda i: (0, 0)),
        out_shape=jax.ShapeDtypeStruct((G, od), jnp.float32),
        scratch_shapes=[
            pltpu.VMEM((G, h), jnp.float32),
            pltpu.VMEM((G, 1), jnp.float32),
        ],
    )(t, st, g, be, batch2d, mW1, mb1, mW2, mb2, mW3, mb3, oW, ob)


def kernel(x, edge_index, batch, W1, b1, g1, be1, W2, b2, g2, be2,
           W3, b3, g3, be3, mW1, mb1, mW2, mb2, mW3, mb3, oW, ob):
    n = x.shape[0]
    src = edge_index[0]
    dst = edge_index[1]

    degp = _sc_degree(dst, n)
    hp1, dinv = _tc_mm1(x, W1, degp)

    t1, st1 = _tc_merge_stats(_sc_message(hp1, src, dst), hp1, dinv,
                              b1.reshape(1, -1))
    hp2 = _tc_bn_mm(t1, st1, g1.reshape(1, -1), be1.reshape(1, -1), W2, dinv)

    t2, st2 = _tc_merge_stats(_sc_message(hp2, src, dst), hp2, dinv,
                              b2.reshape(1, -1))
    hp3 = _tc_bn_mm(t2, st2, g2.reshape(1, -1), be2.reshape(1, -1), W3, dinv)

    t3, st3 = _tc_merge_stats(_sc_message(hp3, src, dst), hp3, dinv,
                              b3.reshape(1, -1))

    out = _tc_final(t3, st3, g3.reshape(1, -1), be3.reshape(1, -1),
                    batch.reshape(n // BN_ROWS, 1, BN_ROWS),
                    mW1, mb1.reshape(1, -1), mW2, mb2.reshape(1, -1),
                    mW3, mb3.reshape(1, -1), oW, ob.reshape(1, -1))
    return out.reshape(G, OUT_DIM, LATENT)


# speedup vs baseline: 22.9201x; 2.1701x over previous
import functools

import jax
import jax.numpy as jnp
from jax import lax
from jax.experimental import pallas as pl
from jax.experimental.pallas import tpu as pltpu
from jax.experimental.pallas import tpu_sc as plsc

NC = 2
NS = 16
NW = NC * NS
KE = 80
G = 64
OUT_DIM = 2
LATENT = 32
BN_ROWS = 1000


def _sc_mesh():
    return plsc.VectorSubcoreMesh(core_axis_name="c", subcore_axis_name="s")


def _row_chunks(n):
    cps = ((n + NS - 1) // NS + 7) // 8 * 8
    last = n - (NS - 1) * cps
    assert 0 < last <= cps
    return cps, last


def _sc_degree(dst, n):
    e = dst.shape[0]
    epw = e // NW
    nch = epw // KE
    cps, last = _row_chunks(n)
    ones = jnp.ones((KE, 128), jnp.float32)
    zeros = jnp.zeros((cps, 128), jnp.float32)

    @functools.partial(
        pl.kernel,
        out_type=jax.ShapeDtypeStruct((NC, n, 128), jnp.float32),
        mesh=_sc_mesh(),
        scratch_types=[
            pltpu.VMEM((1, KE), jnp.int32),
            pltpu.VMEM((KE, 128), jnp.float32),
            pltpu.VMEM_SHARED((n, 128), jnp.float32),
        ],
    )
    def k(dst_hbm, ones_hbm, zeros_hbm, out_hbm, idx_v, ones_v, acc_sh):
        c = lax.axis_index("c")
        s = lax.axis_index("s")
        wid = c * NS + s

        @pl.when(s < NS - 1)
        def _():
            pltpu.sync_copy(zeros_hbm, acc_sh.at[pl.ds(s * cps, cps)])

        @pl.when(s == NS - 1)
        def _():
            pltpu.sync_copy(zeros_hbm.at[pl.ds(0, last)],
                            acc_sh.at[pl.ds((NS - 1) * cps, last)])

        pltpu.sync_copy(ones_hbm, ones_v)
        plsc.subcore_barrier()
        base = wid * epw

        @pl.loop(0, nch)
        def _(ci):
            pltpu.sync_copy(dst_hbm.at[pl.ds(base + ci * KE, KE)], idx_v.at[0])
            pltpu.sync_copy(ones_v, acc_sh.at[idx_v.at[0]], add=True)

        plsc.subcore_barrier()

        @pl.when(s < NS - 1)
        def _():
            pltpu.sync_copy(acc_sh.at[pl.ds(s * cps, cps)],
                            out_hbm.at[c, pl.ds(s * cps, cps)])

        @pl.when(s == NS - 1)
        def _():
            pltpu.sync_copy(acc_sh.at[pl.ds((NS - 1) * cps, last)],
                            out_hbm.at[c, pl.ds((NS - 1) * cps, last)])

    return k(dst, ones, zeros)


def _sc_message(hp, src, dst, n):
    _, h = hp.shape
    e = src.shape[0]
    epw = e // NW
    nch = epw // KE
    cps, last = _row_chunks(n)
    zeros = jnp.zeros((cps, h), jnp.float32)
    NB = 4

    @functools.partial(
        pl.kernel,
        out_type=jax.ShapeDtypeStruct((NC, n, h), jnp.float32),
        mesh=_sc_mesh(),
        scratch_types=[
            pltpu.VMEM((NB, 2, KE), jnp.int32),
            pltpu.VMEM((NB, KE, h), jnp.float32),
            pltpu.VMEM_SHARED((n, h), jnp.float32),
            pltpu.SemaphoreType.DMA((NB,)),
            pltpu.SemaphoreType.DMA((NB,)),
            pltpu.SemaphoreType.DMA((NB,)),
        ],
    )
    def k(hp_hbm, src_hbm, dst_hbm, zeros_hbm, out_hbm, idx_v, rows_v, acc_sh,
          isem, gsem, ssem):
        c = lax.axis_index("c")
        s = lax.axis_index("s")
        wid = c * NS + s

        @pl.when(s < NS - 1)
        def _():
            pltpu.sync_copy(zeros_hbm, acc_sh.at[pl.ds(s * cps, cps)])

        @pl.when(s == NS - 1)
        def _():
            pltpu.sync_copy(zeros_hbm.at[pl.ds(0, last)],
                            acc_sh.at[pl.ds((NS - 1) * cps, last)])

        plsc.subcore_barrier()
        base = wid * epw

        def idx_load(ch, slot):
            pltpu.async_copy(src_hbm.at[pl.ds(base + ch * KE, KE)],
                             idx_v.at[slot, 0], isem.at[slot])
            pltpu.async_copy(dst_hbm.at[pl.ds(base + ch * KE, KE)],
                             idx_v.at[slot, 1], isem.at[slot])

        def idx_wait(ch, slot):
            pltpu.make_async_copy(src_hbm.at[pl.ds(base + ch * KE, KE)],
                                  idx_v.at[slot, 0], isem.at[slot]).wait()
            pltpu.make_async_copy(dst_hbm.at[pl.ds(base + ch * KE, KE)],
                                  idx_v.at[slot, 1], isem.at[slot]).wait()

        def gather(slot):
            pltpu.async_copy(hp_hbm.at[idx_v.at[slot, 0]],
                             rows_v.at[slot], gsem.at[slot])

        def gather_wait(slot):
            pltpu.make_async_copy(hp_hbm.at[idx_v.at[slot, 0]],
                                  rows_v.at[slot], gsem.at[slot]).wait()

        def scatter(slot):
            pltpu.async_copy(rows_v.at[slot], acc_sh.at[idx_v.at[slot, 1]],
                             ssem.at[slot], add=True)

        def scatter_wait(slot):
            pltpu.make_async_copy(rows_v.at[slot],
                                  acc_sh.at[idx_v.at[slot, 1]],
                                  ssem.at[slot]).wait()

        idx_load(0, 0)
        idx_load(1, 1)

        @pl.loop(0, nch)
        def _(ci):
            slot = lax.rem(ci, NB)
            idx_wait(ci, slot)
            gather(slot)

            @pl.when(ci >= 1)
            def _():
                ps = lax.rem(ci + (NB - 1), NB)
                gather_wait(ps)
                scatter(ps)

            @pl.when(ci + 2 < nch)
            def _():
                ns = lax.rem(ci + 2, NB)

                @pl.when(ci >= 2)
                def _():
                    scatter_wait(ns)

                idx_load(ci + 2, ns)

        lt = (nch - 1) % NB
        gather_wait(lt)
        scatter(lt)
        for j in range(NB):
            scatter_wait((nch - NB + j) % NB)

        plsc.subcore_barrier()

        @pl.when(s < NS - 1)
        def _():
            pltpu.sync_copy(acc_sh.at[pl.ds(s * cps, cps)],
                            out_hbm.at[c, pl.ds(s * cps, cps)])

        @pl.when(s == NS - 1)
        def _():
            pltpu.sync_copy(acc_sh.at[pl.ds((NS - 1) * cps, last)],
                            out_hbm.at[c, pl.ds((NS - 1) * cps, last)])

    return k(hp, src, dst, zeros)


def _tc_mm1(x, W, degp):
    n, d = x.shape
    h = W.shape[1]
    nb = n // BN_ROWS

    def body(x_ref, w_ref, degp_ref, hp_ref, dinv_ref):
        dp = degp_ref[...]
        deg = dp[0, :, 0:1] + dp[1, :, 0:1] + 1.0
        dinv = lax.rsqrt(deg)
        hp_ref[...] = dinv * jnp.dot(x_ref[...], w_ref[...],
                                     preferred_element_type=jnp.float32)
        dinv_ref[...] = dinv

    return pl.pallas_call(
        body,
        grid=(nb,),
        in_specs=[
            pl.BlockSpec((BN_ROWS, d), lambda i: (i, 0)),
            pl.BlockSpec((d, h), lambda i: (0, 0)),
            pl.BlockSpec((NC, BN_ROWS, 128), lambda i: (0, i, 0)),
        ],
        out_specs=[
            pl.BlockSpec((BN_ROWS, h), lambda i: (i, 0)),
            pl.BlockSpec((BN_ROWS, 1), lambda i: (i, 0)),
        ],
        out_shape=[
            jax.ShapeDtypeStruct((n, h), jnp.float32),
            jax.ShapeDtypeStruct((n, 1), jnp.float32),
        ],
    )(x, W, degp)


def _tc_merge_stats(P, hp, dinv, b):
    n, h = hp.shape
    nb = n // BN_ROWS

    def body(p_ref, hp_ref, dinv_ref, b_ref, t_ref, st_ref):
        i = pl.program_id(0)
        p = p_ref[...]
        t = dinv_ref[...] * (p[0] + p[1] + hp_ref[...]) + b_ref[...]
        t_ref[...] = t

        @pl.when(i == 0)
        def _():
            st_ref[...] = jnp.zeros_like(st_ref)

        st_ref[...] += jnp.stack([jnp.sum(t, 0), jnp.sum(t * t, 0)])

    return pl.pallas_call(
        body,
        grid=(nb,),
        in_specs=[
            pl.BlockSpec((NC, BN_ROWS, h), lambda i: (0, i, 0)),
            pl.BlockSpec((BN_ROWS, h), lambda i: (i, 0)),
            pl.BlockSpec((BN_ROWS, 1), lambda i: (i, 0)),
            pl.BlockSpec((1, h), lambda i: (0, 0)),
        ],
        out_specs=[
            pl.BlockSpec((BN_ROWS, h), lambda i: (i, 0)),
            pl.BlockSpec((2, h), lambda i: (0, 0)),
        ],
        out_shape=[
            jax.ShapeDtypeStruct((n, h), jnp.float32),
            jax.ShapeDtypeStruct((2, h), jnp.float32),
        ],
    )(P, hp, dinv, b)


def _tc_bn_mm(t, st, g, be, W, dinv):
    n, h = t.shape
    h2 = W.shape[1]
    nb = n // BN_ROWS
    inv_n = 1.0 / n

    def body(t_ref, st_ref, g_ref, be_ref, w_ref, dinv_ref, o_ref):
        st_v = st_ref[...]
        mu = st_v[0:1] * inv_n
        var = st_v[1:2] * inv_n - mu * mu
        y = (t_ref[...] - mu) * lax.rsqrt(var + 1e-5) * g_ref[...] + be_ref[...]
        y = jnp.maximum(y, 0.0)
        o_ref[...] = dinv_ref[...] * jnp.dot(y, w_ref[...],
                                             preferred_element_type=jnp.float32)

    return pl.pallas_call(
        body,
        grid=(nb,),
        in_specs=[
            pl.BlockSpec((BN_ROWS, h), lambda i: (i, 0)),
            pl.BlockSpec((2, h), lambda i: (0, 0)),
            pl.BlockSpec((1, h), lambda i: (0, 0)),
            pl.BlockSpec((1, h), lambda i: (0, 0)),
            pl.BlockSpec((h, h2), lambda i: (0, 0)),
            pl.BlockSpec((BN_ROWS, 1), lambda i: (i, 0)),
        ],
        out_specs=pl.BlockSpec((BN_ROWS, h2), lambda i: (i, 0)),
        out_shape=jax.ShapeDtypeStruct((n, h2), jnp.float32),
    )(t, st, g, be, W, dinv)


def _tc_final(t, st, g, be, batch2d, mW1, mb1, mW2, mb2, mW3, mb3, oW, ob):
    n, h = t.shape
    nb = n // BN_ROWS
    inv_n = 1.0 / n
    h2 = mW2.shape[1]
    h3 = mW3.shape[1]
    od = oW.shape[1]

    def body(t_ref, st_ref, g_ref, be_ref, bat_ref,
             mw1_ref, mb1_ref, mw2_ref, mb2_ref, mw3_ref, mb3_ref,
             ow_ref, ob_ref, out_ref, pool_ref, cnt_ref):
        i = pl.program_id(0)
        st_v = st_ref[...]
        mu = st_v[0:1] * inv_n
        var = st_v[1:2] * inv_n - mu * mu
        y = (t_ref[...] - mu) * lax.rsqrt(var + 1e-5) * g_ref[...] + be_ref[...]
        y = jnp.maximum(y, 0.0)
        seg = bat_ref[0]
        gids = lax.broadcasted_iota(jnp.int32, (G, BN_ROWS), 0)
        onehot = jnp.where(seg == gids, 1.0, 0.0)

        @pl.when(i == 0)
        def _():
            pool_ref[...] = jnp.zeros_like(pool_ref)
            cnt_ref[...] = jnp.zeros_like(cnt_ref)

        pool_ref[...] += jnp.dot(onehot, y, preferred_element_type=jnp.float32)
        cnt_ref[...] += jnp.sum(onehot, axis=1, keepdims=True)

        @pl.when(i == nb - 1)
        def _():
            pooled = pool_ref[...] / jnp.maximum(cnt_ref[...], 1.0)
            z = jnp.maximum(jnp.dot(pooled, mw1_ref[...],
                                    preferred_element_type=jnp.float32) + mb1_ref[...], 0.0)
            z = jnp.maximum(jnp.dot(z, mw2_ref[...],
                                    preferred_element_type=jnp.float32) + mb2_ref[...], 0.0)
            z = jnp.maximum(jnp.dot(z, mw3_ref[...],
                                    preferred_element_type=jnp.float32) + mb3_ref[...], 0.0)
            out_ref[...] = jnp.dot(z, ow_ref[...],
                                   preferred_element_type=jnp.float32) + ob_ref[...]

    return pl.pallas_call(
        body,
        grid=(nb,),
        in_specs=[
            pl.BlockSpec((BN_ROWS, h), lambda i: (i, 0)),
            pl.BlockSpec((2, h), lambda i: (0, 0)),
            pl.BlockSpec((1, h), lambda i: (0, 0)),
            pl.BlockSpec((1, h), lambda i: (0, 0)),
            pl.BlockSpec((1, 1, BN_ROWS), lambda i: (i, 0, 0)),
            pl.BlockSpec((h, h), lambda i: (0, 0)),
            pl.BlockSpec((1, h), lambda i: (0, 0)),
            pl.BlockSpec((h, h2), lambda i: (0, 0)),
            pl.BlockSpec((1, h2), lambda i: (0, 0)),
            pl.BlockSpec((h2, h3), lambda i: (0, 0)),
            pl.BlockSpec((1, h3), lambda i: (0, 0)),
            pl.BlockSpec((h3, od), lambda i: (0, 0)),
            pl.BlockSpec((1, od), lambda i: (0, 0)),
        ],
        out_specs=pl.BlockSpec((G, od), lambda i: (0, 0)),
        out_shape=jax.ShapeDtypeStruct((G, od), jnp.float32),
        scratch_shapes=[
            pltpu.VMEM((G, h), jnp.float32),
            pltpu.VMEM((G, 1), jnp.float32),
        ],
    )(t, st, g, be, batch2d, mW1, mb1, mW2, mb2, mW3, mb3, oW, ob)


def kernel(x, edge_index, batch, W1, b1, g1, be1, W2, b2, g2, be2,
           W3, b3, g3, be3, mW1, mb1, mW2, mb2, mW3, mb3, oW, ob):
    n = x.shape[0]
    src = edge_index[0]
    dst = edge_index[1]

    degp = _sc_degree(dst, n)
    hp1, dinv = _tc_mm1(x, W1, degp)

    t1, st1 = _tc_merge_stats(_sc_message(hp1, src, dst, n), hp1, dinv,
                              b1.reshape(1, -1))
    hp2 = _tc_bn_mm(t1, st1, g1.reshape(1, -1), be1.reshape(1, -1), W2, dinv)

    t2, st2 = _tc_merge_stats(_sc_message(hp2, src, dst, n), hp2, dinv,
                              b2.reshape(1, -1))
    hp3 = _tc_bn_mm(t2, st2, g2.reshape(1, -1), be2.reshape(1, -1), W3, dinv)

    t3, st3 = _tc_merge_stats(_sc_message(hp3, src, dst, n), hp3, dinv,
                              b3.reshape(1, -1))

    out = _tc_final(t3, st3, g3.reshape(1, -1), be3.reshape(1, -1),
                    batch.reshape(n // BN_ROWS, 1, BN_ROWS),
                    mW1, mb1.reshape(1, -1), mW2, mb2.reshape(1, -1),
                    mW3, mb3.reshape(1, -1), oW, ob.reshape(1, -1))
    return out.reshape(G, OUT_DIM, LATENT)


# speedup vs baseline: 28.8932x; 1.2606x over previous
import dataclasses
import functools

import jax
import jax.numpy as jnp
from jax import lax
from jax.experimental import pallas as pl
from jax.experimental.pallas import tpu as pltpu
from jax.experimental.pallas import tpu_sc as plsc

NC = 2
NS = 16
NW = NC * NS
KE = 80
G = 64
OUT_DIM = 2
LATENT = 32
BN_ROWS = 1000


def _sc_mesh():
    return plsc.VectorSubcoreMesh(core_axis_name="c", subcore_axis_name="s")


def _sc_vector_params():
    cp = pltpu.CompilerParams()
    if "needs_layout_passes" in pltpu.CompilerParams.__dataclass_fields__:
        cp = dataclasses.replace(cp, needs_layout_passes=False)
    return cp


def _row_chunks(n):
    cps = ((n + NS - 1) // NS + 7) // 8 * 8
    last = n - (NS - 1) * cps
    assert 0 < last <= cps
    return cps, last


def _sc_degree(dst, n):
    e = dst.shape[0]
    epw = e // NW
    ki = 2000
    nci = epw // ki
    nj = ki // 16

    @functools.partial(
        pl.kernel,
        out_type=jax.ShapeDtypeStruct((NW * n,), jnp.float32),
        mesh=_sc_mesh(),
        compiler_params=_sc_vector_params(),
        scratch_types=[
            pltpu.VMEM((n,), jnp.float32),
            pltpu.VMEM((2 * ki,), jnp.int32),
            pltpu.SemaphoreType.DMA((2,)),
        ],
    )
    def k(dst_hbm, out_hbm, acc_v, buf_v, isem):
        c = lax.axis_index("c")
        s = lax.axis_index("s")
        wid = c * NS + s
        zeros16 = jnp.zeros((16,), jnp.float32)
        ones16 = jnp.ones((16,), jnp.float32)

        @pl.loop(0, n // 16)
        def _(r):
            acc_v[pl.ds(r * 16, 16)] = zeros16

        base = wid * epw
        pltpu.async_copy(dst_hbm.at[pl.ds(base, ki)],
                         buf_v.at[pl.ds(0, ki)], isem.at[0])

        @pl.loop(0, nci)
        def _(ci):
            slot = lax.rem(ci, 2)
            pltpu.make_async_copy(dst_hbm.at[pl.ds(base + ci * ki, ki)],
                                  buf_v.at[pl.ds(slot * ki, ki)],
                                  isem.at[slot]).wait()

            @pl.when(ci + 1 < nci)
            def _():
                nslot = lax.rem(ci + 1, 2)
                pltpu.async_copy(dst_hbm.at[pl.ds(base + (ci + 1) * ki, ki)],
                                 buf_v.at[pl.ds(nslot * ki, ki)],
                                 isem.at[nslot])

            @pl.loop(0, nj)
            def _(j):
                idx = buf_v[pl.ds(slot * ki + j * 16, 16)]
                plsc.addupdate_scatter(acc_v, [idx], ones16)

        pltpu.sync_copy(acc_v, out_hbm.at[pl.ds(wid * n, n)])

    return k(dst)


def _sc_message(hp, src, dst, n):
    _, h = hp.shape
    e = src.shape[0]
    epw = e // NW
    nch = epw // KE
    cps, last = _row_chunks(n)
    zeros = jnp.zeros((cps, h), jnp.float32)
    NB = 4

    @functools.partial(
        pl.kernel,
        out_type=jax.ShapeDtypeStruct((NC, n, h), jnp.float32),
        mesh=_sc_mesh(),
        scratch_types=[
            pltpu.VMEM((NB, 2, KE), jnp.int32),
            pltpu.VMEM((NB, KE, h), jnp.float32),
            pltpu.VMEM_SHARED((n, h), jnp.float32),
            pltpu.SemaphoreType.DMA((NB,)),
            pltpu.SemaphoreType.DMA((NB,)),
            pltpu.SemaphoreType.DMA((NB,)),
        ],
    )
    def k(hp_hbm, src_hbm, dst_hbm, zeros_hbm, out_hbm, idx_v, rows_v, acc_sh,
          isem, gsem, ssem):
        c = lax.axis_index("c")
        s = lax.axis_index("s")
        wid = c * NS + s

        @pl.when(s < NS - 1)
        def _():
            pltpu.sync_copy(zeros_hbm, acc_sh.at[pl.ds(s * cps, cps)])

        @pl.when(s == NS - 1)
        def _():
            pltpu.sync_copy(zeros_hbm.at[pl.ds(0, last)],
                            acc_sh.at[pl.ds((NS - 1) * cps, last)])

        plsc.subcore_barrier()
        base = wid * epw

        def idx_load(ch, slot):
            pltpu.async_copy(src_hbm.at[pl.ds(base + ch * KE, KE)],
                             idx_v.at[slot, 0], isem.at[slot])
            pltpu.async_copy(dst_hbm.at[pl.ds(base + ch * KE, KE)],
                             idx_v.at[slot, 1], isem.at[slot])

        def idx_wait(ch, slot):
            pltpu.make_async_copy(src_hbm.at[pl.ds(base + ch * KE, KE)],
                                  idx_v.at[slot, 0], isem.at[slot]).wait()
            pltpu.make_async_copy(dst_hbm.at[pl.ds(base + ch * KE, KE)],
                                  idx_v.at[slot, 1], isem.at[slot]).wait()

        def gather(slot):
            pltpu.async_copy(hp_hbm.at[idx_v.at[slot, 0]],
                             rows_v.at[slot], gsem.at[slot])

        def gather_wait(slot):
            pltpu.make_async_copy(hp_hbm.at[idx_v.at[slot, 0]],
                                  rows_v.at[slot], gsem.at[slot]).wait()

        def scatter(slot):
            pltpu.async_copy(rows_v.at[slot], acc_sh.at[idx_v.at[slot, 1]],
                             ssem.at[slot], add=True)

        def scatter_wait(slot):
            pltpu.make_async_copy(rows_v.at[slot],
                                  acc_sh.at[idx_v.at[slot, 1]],
                                  ssem.at[slot]).wait()

        idx_load(0, 0)
        idx_load(1, 1)

        @pl.loop(0, nch)
        def _(ci):
            slot = lax.rem(ci, NB)
            idx_wait(ci, slot)
            gather(slot)

            @pl.when(ci >= 1)
            def _():
                ps = lax.rem(ci + (NB - 1), NB)
                gather_wait(ps)
                scatter(ps)

            @pl.when(ci + 2 < nch)
            def _():
                ns = lax.rem(ci + 2, NB)

                @pl.when(ci >= 2)
                def _():
                    scatter_wait(ns)

                idx_load(ci + 2, ns)

        lt = (nch - 1) % NB
        gather_wait(lt)
        scatter(lt)
        for j in range(NB):
            scatter_wait((nch - NB + j) % NB)

        plsc.subcore_barrier()

        @pl.when(s < NS - 1)
        def _():
            pltpu.sync_copy(acc_sh.at[pl.ds(s * cps, cps)],
                            out_hbm.at[c, pl.ds(s * cps, cps)])

        @pl.when(s == NS - 1)
        def _():
            pltpu.sync_copy(acc_sh.at[pl.ds((NS - 1) * cps, last)],
                            out_hbm.at[c, pl.ds((NS - 1) * cps, last)])

    return k(hp, src, dst, zeros)


def _tc_mm1(x, W, degp):
    n, d = x.shape
    h = W.shape[1]
    nb = n // BN_ROWS

    def body(x_ref, w_ref, degp_ref, hp_ref, dinv_ref):
        dp = degp_ref[...].reshape(NW, BN_ROWS)
        deg = lax.dot_general(dp, jnp.ones((NW, 1), jnp.float32),
                              (((0,), (0,)), ((), ())),
                              preferred_element_type=jnp.float32) + 1.0
        dinv = lax.rsqrt(deg)
        hp_ref[...] = dinv * jnp.dot(x_ref[...], w_ref[...],
                                     preferred_element_type=jnp.float32)
        dinv_ref[...] = dinv

    return pl.pallas_call(
        body,
        grid=(nb,),
        in_specs=[
            pl.BlockSpec((BN_ROWS, d), lambda i: (i, 0)),
            pl.BlockSpec((d, h), lambda i: (0, 0)),
            pl.BlockSpec((NW, 1, 1, BN_ROWS), lambda i: (0, i, 0, 0)),
        ],
        out_specs=[
            pl.BlockSpec((BN_ROWS, h), lambda i: (i, 0)),
            pl.BlockSpec((BN_ROWS, 1), lambda i: (i, 0)),
        ],
        out_shape=[
            jax.ShapeDtypeStruct((n, h), jnp.float32),
            jax.ShapeDtypeStruct((n, 1), jnp.float32),
        ],
    )(x, W, degp)


def _tc_merge_stats(P, hp, dinv, b):
    n, h = hp.shape
    nb = n // BN_ROWS

    def body(p_ref, hp_ref, dinv_ref, b_ref, t_ref, st_ref):
        i = pl.program_id(0)
        p = p_ref[...]
        t = dinv_ref[...] * (p[0] + p[1] + hp_ref[...]) + b_ref[...]
        t_ref[...] = t

        @pl.when(i == 0)
        def _():
            st_ref[...] = jnp.zeros_like(st_ref)

        st_ref[...] += jnp.stack([jnp.sum(t, 0), jnp.sum(t * t, 0)])

    return pl.pallas_call(
        body,
        grid=(nb,),
        in_specs=[
            pl.BlockSpec((NC, BN_ROWS, h), lambda i: (0, i, 0)),
            pl.BlockSpec((BN_ROWS, h), lambda i: (i, 0)),
            pl.BlockSpec((BN_ROWS, 1), lambda i: (i, 0)),
            pl.BlockSpec((1, h), lambda i: (0, 0)),
        ],
        out_specs=[
            pl.BlockSpec((BN_ROWS, h), lambda i: (i, 0)),
            pl.BlockSpec((2, h), lambda i: (0, 0)),
        ],
        out_shape=[
            jax.ShapeDtypeStruct((n, h), jnp.float32),
            jax.ShapeDtypeStruct((2, h), jnp.float32),
        ],
    )(P, hp, dinv, b)


def _tc_bn_mm(t, st, g, be, W, dinv):
    n, h = t.shape
    h2 = W.shape[1]
    nb = n // BN_ROWS
    inv_n = 1.0 / n

    def body(t_ref, st_ref, g_ref, be_ref, w_ref, dinv_ref, o_ref):
        st_v = st_ref[...]
        mu = st_v[0:1] * inv_n
        var = st_v[1:2] * inv_n - mu * mu
        y = (t_ref[...] - mu) * lax.rsqrt(var + 1e-5) * g_ref[...] + be_ref[...]
        y = jnp.maximum(y, 0.0)
        o_ref[...] = dinv_ref[...] * jnp.dot(y, w_ref[...],
                                             preferred_element_type=jnp.float32)

    return pl.pallas_call(
        body,
        grid=(nb,),
        in_specs=[
            pl.BlockSpec((BN_ROWS, h), lambda i: (i, 0)),
            pl.BlockSpec((2, h), lambda i: (0, 0)),
            pl.BlockSpec((1, h), lambda i: (0, 0)),
            pl.BlockSpec((1, h), lambda i: (0, 0)),
            pl.BlockSpec((h, h2), lambda i: (0, 0)),
            pl.BlockSpec((BN_ROWS, 1), lambda i: (i, 0)),
        ],
        out_specs=pl.BlockSpec((BN_ROWS, h2), lambda i: (i, 0)),
        out_shape=jax.ShapeDtypeStruct((n, h2), jnp.float32),
    )(t, st, g, be, W, dinv)


def _tc_final(t, st, g, be, batch2d, mW1, mb1, mW2, mb2, mW3, mb3, oW, ob):
    n, h = t.shape
    nb = n // BN_ROWS
    inv_n = 1.0 / n
    h2 = mW2.shape[1]
    h3 = mW3.shape[1]
    od = oW.shape[1]

    def body(t_ref, st_ref, g_ref, be_ref, bat_ref,
             mw1_ref, mb1_ref, mw2_ref, mb2_ref, mw3_ref, mb3_ref,
             ow_ref, ob_ref, out_ref, pool_ref, cnt_ref):
        i = pl.program_id(0)
        st_v = st_ref[...]
        mu = st_v[0:1] * inv_n
        var = st_v[1:2] * inv_n - mu * mu
        y = (t_ref[...] - mu) * lax.rsqrt(var + 1e-5) * g_ref[...] + be_ref[...]
        y = jnp.maximum(y, 0.0)
        seg = bat_ref[0]
        gids = lax.broadcasted_iota(jnp.int32, (G, BN_ROWS), 0)
        onehot = jnp.where(seg == gids, 1.0, 0.0)

        @pl.when(i == 0)
        def _():
            pool_ref[...] = jnp.zeros_like(pool_ref)
            cnt_ref[...] = jnp.zeros_like(cnt_ref)

        pool_ref[...] += jnp.dot(onehot, y, preferred_element_type=jnp.float32)
        cnt_ref[...] += jnp.sum(onehot, axis=1, keepdims=True)

        @pl.when(i == nb - 1)
        def _():
            pooled = pool_ref[...] / jnp.maximum(cnt_ref[...], 1.0)
            z = jnp.maximum(jnp.dot(pooled, mw1_ref[...],
                                    preferred_element_type=jnp.float32) + mb1_ref[...], 0.0)
            z = jnp.maximum(jnp.dot(z, mw2_ref[...],
                                    preferred_element_type=jnp.float32) + mb2_ref[...], 0.0)
            z = jnp.maximum(jnp.dot(z, mw3_ref[...],
                                    preferred_element_type=jnp.float32) + mb3_ref[...], 0.0)
            out_ref[...] = jnp.dot(z, ow_ref[...],
                                   preferred_element_type=jnp.float32) + ob_ref[...]

    return pl.pallas_call(
        body,
        grid=(nb,),
        in_specs=[
            pl.BlockSpec((BN_ROWS, h), lambda i: (i, 0)),
            pl.BlockSpec((2, h), lambda i: (0, 0)),
            pl.BlockSpec((1, h), lambda i: (0, 0)),
            pl.BlockSpec((1, h), lambda i: (0, 0)),
            pl.BlockSpec((1, 1, BN_ROWS), lambda i: (i, 0, 0)),
            pl.BlockSpec((h, h), lambda i: (0, 0)),
            pl.BlockSpec((1, h), lambda i: (0, 0)),
            pl.BlockSpec((h, h2), lambda i: (0, 0)),
            pl.BlockSpec((1, h2), lambda i: (0, 0)),
            pl.BlockSpec((h2, h3), lambda i: (0, 0)),
            pl.BlockSpec((1, h3), lambda i: (0, 0)),
            pl.BlockSpec((h3, od), lambda i: (0, 0)),
            pl.BlockSpec((1, od), lambda i: (0, 0)),
        ],
        out_specs=pl.BlockSpec((G, od), lambda i: (0, 0)),
        out_shape=jax.ShapeDtypeStruct((G, od), jnp.float32),
        scratch_shapes=[
            pltpu.VMEM((G, h), jnp.float32),
            pltpu.VMEM((G, 1), jnp.float32),
        ],
    )(t, st, g, be, batch2d, mW1, mb1, mW2, mb2, mW3, mb3, oW, ob)


def kernel(x, edge_index, batch, W1, b1, g1, be1, W2, b2, g2, be2,
           W3, b3, g3, be3, mW1, mb1, mW2, mb2, mW3, mb3, oW, ob):
    n = x.shape[0]
    src = edge_index[0]
    dst = edge_index[1]

    degp = _sc_degree(dst, n).reshape(NW, n // BN_ROWS, 1, BN_ROWS)
    hp1, dinv = _tc_mm1(x, W1, degp)

    t1, st1 = _tc_merge_stats(_sc_message(hp1, src, dst, n), hp1, dinv,
                              b1.reshape(1, -1))
    hp2 = _tc_bn_mm(t1, st1, g1.reshape(1, -1), be1.reshape(1, -1), W2, dinv)

    t2, st2 = _tc_merge_stats(_sc_message(hp2, src, dst, n), hp2, dinv,
                              b2.reshape(1, -1))
    hp3 = _tc_bn_mm(t2, st2, g2.reshape(1, -1), be2.reshape(1, -1), W3, dinv)

    t3, st3 = _tc_merge_stats(_sc_message(hp3, src, dst, n), hp3, dinv,
                              b3.reshape(1, -1))

    out = _tc_final(t3, st3, g3.reshape(1, -1), be3.reshape(1, -1),
                    batch.reshape(n // BN_ROWS, 1, BN_ROWS),
                    mW1, mb1.reshape(1, -1), mW2, mb2.reshape(1, -1),
                    mW3, mb3.reshape(1, -1), oW, ob.reshape(1, -1))
    return out.reshape(G, OUT_DIM, LATENT)


# speedup vs baseline: 29.2916x; 1.0138x over previous
import dataclasses
import functools

import jax
import jax.numpy as jnp
from jax import lax
from jax.experimental import pallas as pl
from jax.experimental.pallas import tpu as pltpu
from jax.experimental.pallas import tpu_sc as plsc

NC = 2
NS = 16
NW = NC * NS
KE = 80
G = 64
OUT_DIM = 2
LATENT = 32
BN_ROWS = 1000


def _sc_mesh():
    return plsc.VectorSubcoreMesh(core_axis_name="c", subcore_axis_name="s")


def _sc_vector_params():
    cp = pltpu.CompilerParams()
    if "needs_layout_passes" in pltpu.CompilerParams.__dataclass_fields__:
        cp = dataclasses.replace(cp, needs_layout_passes=False)
    return cp


def _row_chunks(n):
    cps = ((n + NS - 1) // NS + 7) // 8 * 8
    last = n - (NS - 1) * cps
    assert 0 < last <= cps
    return cps, last


def _sc_degree(dst, n):
    e = dst.shape[0]
    epw = e // NW
    ki = 2000
    nci = epw // ki
    nj = ki // 16

    @functools.partial(
        pl.kernel,
        out_type=jax.ShapeDtypeStruct((NW * n,), jnp.float32),
        mesh=_sc_mesh(),
        compiler_params=_sc_vector_params(),
        scratch_types=[
            pltpu.VMEM((n,), jnp.float32),
            pltpu.VMEM((2 * ki,), jnp.int32),
            pltpu.SemaphoreType.DMA((2,)),
        ],
    )
    def k(dst_hbm, out_hbm, acc_v, buf_v, isem):
        c = lax.axis_index("c")
        s = lax.axis_index("s")
        wid = c * NS + s
        zeros16 = jnp.zeros((16,), jnp.float32)
        ones16 = jnp.ones((16,), jnp.float32)

        @pl.loop(0, n // 16)
        def _(r):
            acc_v[pl.ds(r * 16, 16)] = zeros16

        base = wid * epw
        pltpu.async_copy(dst_hbm.at[pl.ds(base, ki)],
                         buf_v.at[pl.ds(0, ki)], isem.at[0])

        @pl.loop(0, nci)
        def _(ci):
            slot = lax.rem(ci, 2)
            pltpu.make_async_copy(dst_hbm.at[pl.ds(base + ci * ki, ki)],
                                  buf_v.at[pl.ds(slot * ki, ki)],
                                  isem.at[slot]).wait()

            @pl.when(ci + 1 < nci)
            def _():
                nslot = lax.rem(ci + 1, 2)
                pltpu.async_copy(dst_hbm.at[pl.ds(base + (ci + 1) * ki, ki)],
                                 buf_v.at[pl.ds(nslot * ki, ki)],
                                 isem.at[nslot])

            @pl.loop(0, nj)
            def _(j):
                idx = buf_v[pl.ds(slot * ki + j * 16, 16)]
                plsc.addupdate_scatter(acc_v, [idx], ones16)

        pltpu.sync_copy(acc_v, out_hbm.at[pl.ds(wid * n, n)])

    return k(dst)


def _sc_message(hp, src, dst, n):
    _, h = hp.shape
    e = src.shape[0]
    epw = e // NW
    nch = epw // KE
    cps, last = _row_chunks(n)
    zeros = jnp.zeros((cps, h), jnp.float32)
    NB = 4

    @functools.partial(
        pl.kernel,
        out_type=jax.ShapeDtypeStruct((NC, n, h), jnp.float32),
        mesh=_sc_mesh(),
        scratch_types=[
            pltpu.VMEM((NB, 2, KE), jnp.int32),
            pltpu.VMEM((NB, KE, h), jnp.float32),
            pltpu.VMEM_SHARED((n, h), jnp.float32),
            pltpu.SemaphoreType.DMA((NB,)),
            pltpu.SemaphoreType.DMA((NB,)),
            pltpu.SemaphoreType.DMA((NB,)),
        ],
    )
    def k(hp_hbm, src_hbm, dst_hbm, zeros_hbm, out_hbm, idx_v, rows_v, acc_sh,
          isem, gsem, ssem):
        c = lax.axis_index("c")
        s = lax.axis_index("s")
        wid = c * NS + s

        @pl.when(s < NS - 1)
        def _():
            pltpu.sync_copy(zeros_hbm, acc_sh.at[pl.ds(s * cps, cps)])

        @pl.when(s == NS - 1)
        def _():
            pltpu.sync_copy(zeros_hbm.at[pl.ds(0, last)],
                            acc_sh.at[pl.ds((NS - 1) * cps, last)])

        plsc.subcore_barrier()
        base = wid * epw

        def idx_load(ch, slot):
            pltpu.async_copy(src_hbm.at[pl.ds(base + ch * KE, KE)],
                             idx_v.at[slot, 0], isem.at[slot])
            pltpu.async_copy(dst_hbm.at[pl.ds(base + ch * KE, KE)],
                             idx_v.at[slot, 1], isem.at[slot])

        def idx_wait(ch, slot):
            pltpu.make_async_copy(src_hbm.at[pl.ds(base + ch * KE, KE)],
                                  idx_v.at[slot, 0], isem.at[slot]).wait()
            pltpu.make_async_copy(dst_hbm.at[pl.ds(base + ch * KE, KE)],
                                  idx_v.at[slot, 1], isem.at[slot]).wait()

        def gather(slot):
            pltpu.async_copy(hp_hbm.at[idx_v.at[slot, 0]],
                             rows_v.at[slot], gsem.at[slot])

        def gather_wait(slot):
            pltpu.make_async_copy(hp_hbm.at[idx_v.at[slot, 0]],
                                  rows_v.at[slot], gsem.at[slot]).wait()

        def scatter(slot):
            pltpu.async_copy(rows_v.at[slot], acc_sh.at[idx_v.at[slot, 1]],
                             ssem.at[slot], add=True)

        def scatter_wait(slot):
            pltpu.make_async_copy(rows_v.at[slot],
                                  acc_sh.at[idx_v.at[slot, 1]],
                                  ssem.at[slot]).wait()

        idx_load(0, 0)
        idx_load(1, 1)

        @pl.loop(0, nch)
        def _(ci):
            slot = lax.rem(ci, NB)
            idx_wait(ci, slot)
            gather(slot)

            @pl.when(ci >= 1)
            def _():
                ps = lax.rem(ci + (NB - 1), NB)
                gather_wait(ps)
                scatter(ps)

            @pl.when(ci + 2 < nch)
            def _():
                ns = lax.rem(ci + 2, NB)

                @pl.when(ci >= 2)
                def _():
                    scatter_wait(ns)

                idx_load(ci + 2, ns)

        lt = (nch - 1) % NB
        gather_wait(lt)
        scatter(lt)
        for j in range(NB):
            scatter_wait((nch - NB + j) % NB)

        plsc.subcore_barrier()

        @pl.when(s < NS - 1)
        def _():
            pltpu.sync_copy(acc_sh.at[pl.ds(s * cps, cps)],
                            out_hbm.at[c, pl.ds(s * cps, cps)])

        @pl.when(s == NS - 1)
        def _():
            pltpu.sync_copy(acc_sh.at[pl.ds((NS - 1) * cps, last)],
                            out_hbm.at[c, pl.ds((NS - 1) * cps, last)])

    return k(hp, src, dst, zeros)


def _tc_mm1(x, W, degp):
    n, d = x.shape
    h = W.shape[1]
    nb = n // BN_ROWS

    def body(x_ref, w_ref, degp_ref, hp_ref, dinv_ref):
        dp = degp_ref[...].reshape(NW, BN_ROWS)
        deg = lax.dot_general(dp, jnp.ones((NW, 1), jnp.float32),
                              (((0,), (0,)), ((), ())),
                              preferred_element_type=jnp.float32) + 1.0
        dinv = lax.rsqrt(deg)
        hp_ref[...] = dinv * jnp.dot(x_ref[...], w_ref[...],
                                     preferred_element_type=jnp.float32)
        dinv_ref[...] = dinv

    return pl.pallas_call(
        body,
        grid=(nb,),
        in_specs=[
            pl.BlockSpec((BN_ROWS, d), lambda i: (i, 0)),
            pl.BlockSpec((d, h), lambda i: (0, 0)),
            pl.BlockSpec((NW, 1, 1, BN_ROWS), lambda i: (0, i, 0, 0)),
        ],
        out_specs=[
            pl.BlockSpec((BN_ROWS, h), lambda i: (i, 0)),
            pl.BlockSpec((BN_ROWS, 1), lambda i: (i, 0)),
        ],
        out_shape=[
            jax.ShapeDtypeStruct((n, h), jnp.float32),
            jax.ShapeDtypeStruct((n, 1), jnp.float32),
        ],
    )(x, W, degp)


def _tc_layer(P, hp, dinv, b, g, be, W):
    n, h = hp.shape
    h2 = W.shape[1]
    nb = n // BN_ROWS
    inv_n = 1.0 / n

    def body(p_ref, hp_ref, dinv_ref, b_ref, g_ref, be_ref, w_ref,
             o_ref, t_scr, st_scr):
        ph = pl.program_id(0)
        j = pl.program_id(1)

        @pl.when(ph == 0)
        def _():
            p = p_ref[...]
            t = dinv_ref[...] * (p[0] + p[1] + hp_ref[...]) + b_ref[...]
            t_scr[pl.ds(j * BN_ROWS, BN_ROWS), :] = t

            @pl.when(j == 0)
            def _():
                st_scr[...] = jnp.zeros_like(st_scr)

            st_scr[...] += jnp.stack([jnp.sum(t, 0), jnp.sum(t * t, 0)])

        @pl.when(ph == 1)
        def _():
            st_v = st_scr[...]
            mu = st_v[0:1] * inv_n
            var = st_v[1:2] * inv_n - mu * mu
            t = t_scr[pl.ds(j * BN_ROWS, BN_ROWS), :]
            y = (t - mu) * lax.rsqrt(var + 1e-5) * g_ref[...] + be_ref[...]
            y = jnp.maximum(y, 0.0)
            o_ref[...] = dinv_ref[...] * jnp.dot(
                y, w_ref[...], preferred_element_type=jnp.float32)

    return pl.pallas_call(
        body,
        grid=(2, nb),
        in_specs=[
            pl.BlockSpec((NC, BN_ROWS, h),
                         lambda p, j: (0, jnp.where(p == 0, j, 0), 0)),
            pl.BlockSpec((BN_ROWS, h),
                         lambda p, j: (jnp.where(p == 0, j, 0), 0)),
            pl.BlockSpec((BN_ROWS, 1), lambda p, j: (j, 0)),
            pl.BlockSpec((1, h), lambda p, j: (0, 0)),
            pl.BlockSpec((1, h), lambda p, j: (0, 0)),
            pl.BlockSpec((1, h), lambda p, j: (0, 0)),
            pl.BlockSpec((h, h2), lambda p, j: (0, 0)),
        ],
        out_specs=pl.BlockSpec((BN_ROWS, h2),
                               lambda p, j: (jnp.where(p == 0, 0, j), 0)),
        out_shape=jax.ShapeDtypeStruct((n, h2), jnp.float32),
        scratch_shapes=[
            pltpu.VMEM((n, h), jnp.float32),
            pltpu.VMEM((2, h), jnp.float32),
        ],
    )(P, hp, dinv, b, g, be, W)


def _tc_last(P, hp, dinv, b, g, be, batch3d,
             mW1, mb1, mW2, mb2, mW3, mb3, oW, ob):
    n, h = hp.shape
    nb = n // BN_ROWS
    inv_n = 1.0 / n
    h2 = mW2.shape[1]
    h3 = mW3.shape[1]
    od = oW.shape[1]

    def body(p_ref, hp_ref, dinv_ref, b_ref, g_ref, be_ref, bat_ref,
             mw1_ref, mb1_ref, mw2_ref, mb2_ref, mw3_ref, mb3_ref,
             ow_ref, ob_ref, out_ref, t_scr, st_scr, pool_ref, cnt_ref):
        ph = pl.program_id(0)
        j = pl.program_id(1)

        @pl.when(ph == 0)
        def _():
            p = p_ref[...]
            t = dinv_ref[...] * (p[0] + p[1] + hp_ref[...]) + b_ref[...]
            t_scr[pl.ds(j * BN_ROWS, BN_ROWS), :] = t

            @pl.when(j == 0)
            def _():
                st_scr[...] = jnp.zeros_like(st_scr)
                pool_ref[...] = jnp.zeros_like(pool_ref)
                cnt_ref[...] = jnp.zeros_like(cnt_ref)

            st_scr[...] += jnp.stack([jnp.sum(t, 0), jnp.sum(t * t, 0)])

        @pl.when(ph == 1)
        def _():
            st_v = st_scr[...]
            mu = st_v[0:1] * inv_n
            var = st_v[1:2] * inv_n - mu * mu
            t = t_scr[pl.ds(j * BN_ROWS, BN_ROWS), :]
            y = (t - mu) * lax.rsqrt(var + 1e-5) * g_ref[...] + be_ref[...]
            y = jnp.maximum(y, 0.0)
            seg = bat_ref[0]
            gids = lax.broadcasted_iota(jnp.int32, (G, BN_ROWS), 0)
            onehot = jnp.where(seg == gids, 1.0, 0.0)
            pool_ref[...] += jnp.dot(onehot, y,
                                     preferred_element_type=jnp.float32)
            cnt_ref[...] += jnp.sum(onehot, axis=1, keepdims=True)

            @pl.when(j == nb - 1)
            def _():
                pooled = pool_ref[...] / jnp.maximum(cnt_ref[...], 1.0)
                z = jnp.maximum(jnp.dot(pooled, mw1_ref[...],
                                        preferred_element_type=jnp.float32)
                                + mb1_ref[...], 0.0)
                z = jnp.maximum(jnp.dot(z, mw2_ref[...],
                                        preferred_element_type=jnp.float32)
                                + mb2_ref[...], 0.0)
                z = jnp.maximum(jnp.dot(z, mw3_ref[...],
                                        preferred_element_type=jnp.float32)
                                + mb3_ref[...], 0.0)
                out_ref[...] = jnp.dot(z, ow_ref[...],
                                       preferred_element_type=jnp.float32) + ob_ref[...]

    return pl.pallas_call(
        body,
        grid=(2, nb),
        in_specs=[
            pl.BlockSpec((NC, BN_ROWS, h),
                         lambda p, j: (0, jnp.where(p == 0, j, 0), 0)),
            pl.BlockSpec((BN_ROWS, h),
                         lambda p, j: (jnp.where(p == 0, j, 0), 0)),
            pl.BlockSpec((BN_ROWS, 1), lambda p, j: (j, 0)),
            pl.BlockSpec((1, h), lambda p, j: (0, 0)),
            pl.BlockSpec((1, h), lambda p, j: (0, 0)),
            pl.BlockSpec((1, h), lambda p, j: (0, 0)),
            pl.BlockSpec((1, 1, BN_ROWS), lambda p, j: (j, 0, 0)),
            pl.BlockSpec((h, h), lambda p, j: (0, 0)),
            pl.BlockSpec((1, h), lambda p, j: (0, 0)),
            pl.BlockSpec((h, h2), lambda p, j: (0, 0)),
            pl.BlockSpec((1, h2), lambda p, j: (0, 0)),
            pl.BlockSpec((h2, h3), lambda p, j: (0, 0)),
            pl.BlockSpec((1, h3), lambda p, j: (0, 0)),
            pl.BlockSpec((h3, od), lambda p, j: (0, 0)),
            pl.BlockSpec((1, od), lambda p, j: (0, 0)),
        ],
        out_specs=pl.BlockSpec((G, od), lambda p, j: (0, 0)),
        out_shape=jax.ShapeDtypeStruct((G, od), jnp.float32),
        scratch_shapes=[
            pltpu.VMEM((n, h), jnp.float32),
            pltpu.VMEM((2, h), jnp.float32),
            pltpu.VMEM((G, h), jnp.float32),
            pltpu.VMEM((G, 1), jnp.float32),
        ],
    )(P, hp, dinv, b, g, be, batch3d, mW1, mb1, mW2, mb2, mW3, mb3, oW, ob)


def kernel(x, edge_index, batch, W1, b1, g1, be1, W2, b2, g2, be2,
           W3, b3, g3, be3, mW1, mb1, mW2, mb2, mW3, mb3, oW, ob):
    n = x.shape[0]
    src = edge_index[0]
    dst = edge_index[1]

    degp = _sc_degree(dst, n).reshape(NW, n // BN_ROWS, 1, BN_ROWS)
    hp1, dinv = _tc_mm1(x, W1, degp)

    hp2 = _tc_layer(_sc_message(hp1, src, dst, n), hp1, dinv,
                    b1.reshape(1, -1), g1.reshape(1, -1), be1.reshape(1, -1),
                    W2)
    hp3 = _tc_layer(_sc_message(hp2, src, dst, n), hp2, dinv,
                    b2.reshape(1, -1), g2.reshape(1, -1), be2.reshape(1, -1),
                    W3)
    out = _tc_last(_sc_message(hp3, src, dst, n), hp3, dinv,
                   b3.reshape(1, -1), g3.reshape(1, -1), be3.reshape(1, -1),
                   batch.reshape(n // BN_ROWS, 1, BN_ROWS),
                   mW1, mb1.reshape(1, -1), mW2, mb2.reshape(1, -1),
                   mW3, mb3.reshape(1, -1), oW, ob.reshape(1, -1))
    return out.reshape(G, OUT_DIM, LATENT)


# speedup vs baseline: 29.2974x; 1.0002x over previous
import dataclasses
import functools

import jax
import jax.numpy as jnp
from jax import lax
from jax.experimental import pallas as pl
from jax.experimental.pallas import tpu as pltpu
from jax.experimental.pallas import tpu_sc as plsc

NC = 2
NS = 16
NW = NC * NS
KE = 80
G = 64
OUT_DIM = 2
LATENT = 32
BN_ROWS = 1000


def _sc_mesh():
    return plsc.VectorSubcoreMesh(core_axis_name="c", subcore_axis_name="s")


def _sc_vector_params():
    cp = pltpu.CompilerParams()
    if "needs_layout_passes" in pltpu.CompilerParams.__dataclass_fields__:
        cp = dataclasses.replace(cp, needs_layout_passes=False)
    return cp


def _row_chunks(n):
    cps = ((n + NS - 1) // NS + 7) // 8 * 8
    last = n - (NS - 1) * cps
    assert 0 < last <= cps
    return cps, last


def _sc_degree(dst, n):
    e = dst.shape[0]
    epw = e // NW
    ki = 2000
    nci = epw // ki
    nj = ki // 16

    @functools.partial(
        pl.kernel,
        out_type=jax.ShapeDtypeStruct((NW * n,), jnp.float32),
        mesh=_sc_mesh(),
        compiler_params=_sc_vector_params(),
        scratch_types=[
            pltpu.VMEM((n,), jnp.float32),
            pltpu.VMEM((2 * ki,), jnp.int32),
            pltpu.SemaphoreType.DMA((2,)),
        ],
    )
    def k(dst_hbm, out_hbm, acc_v, buf_v, isem):
        c = lax.axis_index("c")
        s = lax.axis_index("s")
        wid = c * NS + s
        zeros16 = jnp.zeros((16,), jnp.float32)
        ones16 = jnp.ones((16,), jnp.float32)

        @pl.loop(0, n // 16)
        def _(r):
            acc_v[pl.ds(r * 16, 16)] = zeros16

        base = wid * epw
        pltpu.async_copy(dst_hbm.at[pl.ds(base, ki)],
                         buf_v.at[pl.ds(0, ki)], isem.at[0])

        @pl.loop(0, nci)
        def _(ci):
            slot = lax.rem(ci, 2)
            pltpu.make_async_copy(dst_hbm.at[pl.ds(base + ci * ki, ki)],
                                  buf_v.at[pl.ds(slot * ki, ki)],
                                  isem.at[slot]).wait()

            @pl.when(ci + 1 < nci)
            def _():
                nslot = lax.rem(ci + 1, 2)
                pltpu.async_copy(dst_hbm.at[pl.ds(base + (ci + 1) * ki, ki)],
                                 buf_v.at[pl.ds(nslot * ki, ki)],
                                 isem.at[nslot])

            @pl.loop(0, nj)
            def _(j):
                idx = buf_v[pl.ds(slot * ki + j * 16, 16)]
                plsc.addupdate_scatter(acc_v, [idx], ones16)

        pltpu.sync_copy(acc_v, out_hbm.at[pl.ds(wid * n, n)])

    return k(dst)


def _sc_message(hp, src, dst, n):
    _, h = hp.shape
    e = src.shape[0]
    epw = e // NW
    nch = epw // KE
    cps, last = _row_chunks(n)
    zeros = jnp.zeros((cps, h), jnp.float32)
    NB = 4

    @functools.partial(
        pl.kernel,
        out_type=jax.ShapeDtypeStruct((NC, n, h), jnp.float32),
        mesh=_sc_mesh(),
        scratch_types=[
            pltpu.VMEM((NB, 2, KE), jnp.int32),
            pltpu.VMEM((NB, KE, h), jnp.float32),
            pltpu.VMEM_SHARED((n, h), jnp.float32),
            pltpu.SemaphoreType.DMA((NB,)),
            pltpu.SemaphoreType.DMA((NB,)),
            pltpu.SemaphoreType.DMA((NB,)),
        ],
    )
    def k(hp_hbm, src_hbm, dst_hbm, zeros_hbm, out_hbm, idx_v, rows_v, acc_sh,
          isem, gsem, ssem):
        c = lax.axis_index("c")
        s = lax.axis_index("s")
        wid = c * NS + s

        @pl.when(s < NS - 1)
        def _():
            pltpu.sync_copy(zeros_hbm, acc_sh.at[pl.ds(s * cps, cps)])

        @pl.when(s == NS - 1)
        def _():
            pltpu.sync_copy(zeros_hbm.at[pl.ds(0, last)],
                            acc_sh.at[pl.ds((NS - 1) * cps, last)])

        plsc.subcore_barrier()
        base = wid * epw

        def idx_load(ch, slot):
            pltpu.async_copy(src_hbm.at[pl.ds(base + ch * KE, KE)],
                             idx_v.at[slot, 0], isem.at[slot])
            pltpu.async_copy(dst_hbm.at[pl.ds(base + ch * KE, KE)],
                             idx_v.at[slot, 1], isem.at[slot])

        def idx_wait(ch, slot):
            pltpu.make_async_copy(src_hbm.at[pl.ds(base + ch * KE, KE)],
                                  idx_v.at[slot, 0], isem.at[slot]).wait()
            pltpu.make_async_copy(dst_hbm.at[pl.ds(base + ch * KE, KE)],
                                  idx_v.at[slot, 1], isem.at[slot]).wait()

        def gather(slot):
            pltpu.async_copy(hp_hbm.at[idx_v.at[slot, 0]],
                             rows_v.at[slot], gsem.at[slot])

        def gather_wait(slot):
            pltpu.make_async_copy(hp_hbm.at[idx_v.at[slot, 0]],
                                  rows_v.at[slot], gsem.at[slot]).wait()

        def scatter(slot):
            pltpu.async_copy(rows_v.at[slot], acc_sh.at[idx_v.at[slot, 1]],
                             ssem.at[slot], add=True)

        def scatter_wait(slot):
            pltpu.make_async_copy(rows_v.at[slot],
                                  acc_sh.at[idx_v.at[slot, 1]],
                                  ssem.at[slot]).wait()

        idx_load(0, 0)
        idx_load(1, 1)

        @pl.loop(0, nch)
        def _(ci):
            slot = lax.rem(ci, NB)
            idx_wait(ci, slot)
            gather(slot)

            @pl.when(ci >= 1)
            def _():
                ps = lax.rem(ci + (NB - 1), NB)
                gather_wait(ps)
                scatter(ps)

            @pl.when(ci + 2 < nch)
            def _():
                ns = lax.rem(ci + 2, NB)

                @pl.when(ci >= NB - 2)
                def _():
                    scatter_wait(ns)

                idx_load(ci + 2, ns)

        lt = (nch - 1) % NB
        gather_wait(lt)
        scatter(lt)
        for j in range(NB):
            scatter_wait((nch - NB + j) % NB)

        plsc.subcore_barrier()

        @pl.when(s < NS - 1)
        def _():
            pltpu.sync_copy(acc_sh.at[pl.ds(s * cps, cps)],
                            out_hbm.at[c, pl.ds(s * cps, cps)])

        @pl.when(s == NS - 1)
        def _():
            pltpu.sync_copy(acc_sh.at[pl.ds((NS - 1) * cps, last)],
                            out_hbm.at[c, pl.ds((NS - 1) * cps, last)])

    return k(hp, src, dst, zeros)


def _tc_mm1(x, W, degp):
    n, d = x.shape
    h = W.shape[1]
    nb = n // BN_ROWS

    def body(x_ref, w_ref, degp_ref, hp_ref, dinv_ref):
        dp = degp_ref[...].reshape(NW, BN_ROWS)
        deg = lax.dot_general(dp, jnp.ones((NW, 1), jnp.float32),
                              (((0,), (0,)), ((), ())),
                              preferred_element_type=jnp.float32) + 1.0
        dinv = lax.rsqrt(deg)
        hp_ref[...] = dinv * jnp.dot(x_ref[...], w_ref[...],
                                     preferred_element_type=jnp.float32)
        dinv_ref[...] = dinv

    return pl.pallas_call(
        body,
        grid=(nb,),
        in_specs=[
            pl.BlockSpec((BN_ROWS, d), lambda i: (i, 0)),
            pl.BlockSpec((d, h), lambda i: (0, 0)),
            pl.BlockSpec((NW, 1, 1, BN_ROWS), lambda i: (0, i, 0, 0)),
        ],
        out_specs=[
            pl.BlockSpec((BN_ROWS, h), lambda i: (i, 0)),
            pl.BlockSpec((BN_ROWS, 1), lambda i: (i, 0)),
        ],
        out_shape=[
            jax.ShapeDtypeStruct((n, h), jnp.float32),
            jax.ShapeDtypeStruct((n, 1), jnp.float32),
        ],
    )(x, W, degp)


def _tc_layer(P, hp, dinv, b, g, be, W):
    n, h = hp.shape
    h2 = W.shape[1]
    nb = n // BN_ROWS
    inv_n = 1.0 / n

    def body(p_ref, hp_ref, dinv_ref, b_ref, g_ref, be_ref, w_ref,
             o_ref, t_scr, st_scr):
        ph = pl.program_id(0)
        j = pl.program_id(1)

        @pl.when(ph == 0)
        def _():
            p = p_ref[...]
            t = dinv_ref[...] * (p[0] + p[1] + hp_ref[...]) + b_ref[...]
            t_scr[pl.ds(j * BN_ROWS, BN_ROWS), :] = t

            @pl.when(j == 0)
            def _():
                st_scr[...] = jnp.zeros_like(st_scr)

            st_scr[...] += jnp.stack([jnp.sum(t, 0), jnp.sum(t * t, 0)])

        @pl.when(ph == 1)
        def _():
            st_v = st_scr[...]
            mu = st_v[0:1] * inv_n
            var = st_v[1:2] * inv_n - mu * mu
            t = t_scr[pl.ds(j * BN_ROWS, BN_ROWS), :]
            y = (t - mu) * lax.rsqrt(var + 1e-5) * g_ref[...] + be_ref[...]
            y = jnp.maximum(y, 0.0)
            o_ref[...] = dinv_ref[...] * jnp.dot(
                y, w_ref[...], preferred_element_type=jnp.float32)

    return pl.pallas_call(
        body,
        grid=(2, nb),
        in_specs=[
            pl.BlockSpec((NC, BN_ROWS, h),
                         lambda p, j: (0, jnp.where(p == 0, j, 0), 0)),
            pl.BlockSpec((BN_ROWS, h),
                         lambda p, j: (jnp.where(p == 0, j, 0), 0)),
            pl.BlockSpec((BN_ROWS, 1), lambda p, j: (j, 0)),
            pl.BlockSpec((1, h), lambda p, j: (0, 0)),
            pl.BlockSpec((1, h), lambda p, j: (0, 0)),
            pl.BlockSpec((1, h), lambda p, j: (0, 0)),
            pl.BlockSpec((h, h2), lambda p, j: (0, 0)),
        ],
        out_specs=pl.BlockSpec((BN_ROWS, h2),
                               lambda p, j: (jnp.where(p == 0, 0, j), 0)),
        out_shape=jax.ShapeDtypeStruct((n, h2), jnp.float32),
        scratch_shapes=[
            pltpu.VMEM((n, h), jnp.float32),
            pltpu.VMEM((2, h), jnp.float32),
        ],
    )(P, hp, dinv, b, g, be, W)


def _tc_last(P, hp, dinv, b, g, be, batch3d,
             mW1, mb1, mW2, mb2, mW3, mb3, oW, ob):
    n, h = hp.shape
    nb = n // BN_ROWS
    inv_n = 1.0 / n
    h2 = mW2.shape[1]
    h3 = mW3.shape[1]
    od = oW.shape[1]

    def body(p_ref, hp_ref, dinv_ref, b_ref, g_ref, be_ref, bat_ref,
             mw1_ref, mb1_ref, mw2_ref, mb2_ref, mw3_ref, mb3_ref,
             ow_ref, ob_ref, out_ref, t_scr, st_scr, pool_ref, cnt_ref):
        ph = pl.program_id(0)
        j = pl.program_id(1)

        @pl.when(ph == 0)
        def _():
            p = p_ref[...]
            t = dinv_ref[...] * (p[0] + p[1] + hp_ref[...]) + b_ref[...]
            t_scr[pl.ds(j * BN_ROWS, BN_ROWS), :] = t

            @pl.when(j == 0)
            def _():
                st_scr[...] = jnp.zeros_like(st_scr)
                pool_ref[...] = jnp.zeros_like(pool_ref)
                cnt_ref[...] = jnp.zeros_like(cnt_ref)

            st_scr[...] += jnp.stack([jnp.sum(t, 0), jnp.sum(t * t, 0)])

        @pl.when(ph == 1)
        def _():
            st_v = st_scr[...]
            mu = st_v[0:1] * inv_n
            var = st_v[1:2] * inv_n - mu * mu
            t = t_scr[pl.ds(j * BN_ROWS, BN_ROWS), :]
            y = (t - mu) * lax.rsqrt(var + 1e-5) * g_ref[...] + be_ref[...]
            y = jnp.maximum(y, 0.0)
            seg = bat_ref[0]
            gids = lax.broadcasted_iota(jnp.int32, (G, BN_ROWS), 0)
            onehot = jnp.where(seg == gids, 1.0, 0.0)
            pool_ref[...] += jnp.dot(onehot, y,
                                     preferred_element_type=jnp.float32)
            cnt_ref[...] += jnp.sum(onehot, axis=1, keepdims=True)

            @pl.when(j == nb - 1)
            def _():
                pooled = pool_ref[...] / jnp.maximum(cnt_ref[...], 1.0)
                z = jnp.maximum(jnp.dot(pooled, mw1_ref[...],
                                        preferred_element_type=jnp.float32)
                                + mb1_ref[...], 0.0)
                z = jnp.maximum(jnp.dot(z, mw2_ref[...],
                                        preferred_element_type=jnp.float32)
                                + mb2_ref[...], 0.0)
                z = jnp.maximum(jnp.dot(z, mw3_ref[...],
                                        preferred_element_type=jnp.float32)
                                + mb3_ref[...], 0.0)
                out_ref[...] = jnp.dot(z, ow_ref[...],
                                       preferred_element_type=jnp.float32) + ob_ref[...]

    return pl.pallas_call(
        body,
        grid=(2, nb),
        in_specs=[
            pl.BlockSpec((NC, BN_ROWS, h),
                         lambda p, j: (0, jnp.where(p == 0, j, 0), 0)),
            pl.BlockSpec((BN_ROWS, h),
                         lambda p, j: (jnp.where(p == 0, j, 0), 0)),
            pl.BlockSpec((BN_ROWS, 1), lambda p, j: (j, 0)),
            pl.BlockSpec((1, h), lambda p, j: (0, 0)),
            pl.BlockSpec((1, h), lambda p, j: (0, 0)),
            pl.BlockSpec((1, h), lambda p, j: (0, 0)),
            pl.BlockSpec((1, 1, BN_ROWS), lambda p, j: (j, 0, 0)),
            pl.BlockSpec((h, h), lambda p, j: (0, 0)),
            pl.BlockSpec((1, h), lambda p, j: (0, 0)),
            pl.BlockSpec((h, h2), lambda p, j: (0, 0)),
            pl.BlockSpec((1, h2), lambda p, j: (0, 0)),
            pl.BlockSpec((h2, h3), lambda p, j: (0, 0)),
            pl.BlockSpec((1, h3), lambda p, j: (0, 0)),
            pl.BlockSpec((h3, od), lambda p, j: (0, 0)),
            pl.BlockSpec((1, od), lambda p, j: (0, 0)),
        ],
        out_specs=pl.BlockSpec((G, od), lambda p, j: (0, 0)),
        out_shape=jax.ShapeDtypeStruct((G, od), jnp.float32),
        scratch_shapes=[
            pltpu.VMEM((n, h), jnp.float32),
            pltpu.VMEM((2, h), jnp.float32),
            pltpu.VMEM((G, h), jnp.float32),
            pltpu.VMEM((G, 1), jnp.float32),
        ],
    )(P, hp, dinv, b, g, be, batch3d, mW1, mb1, mW2, mb2, mW3, mb3, oW, ob)


def kernel(x, edge_index, batch, W1, b1, g1, be1, W2, b2, g2, be2,
           W3, b3, g3, be3, mW1, mb1, mW2, mb2, mW3, mb3, oW, ob):
    n = x.shape[0]
    src = edge_index[0]
    dst = edge_index[1]

    degp = _sc_degree(dst, n).reshape(NW, n // BN_ROWS, 1, BN_ROWS)
    hp1, dinv = _tc_mm1(x, W1, degp)

    hp2 = _tc_layer(_sc_message(hp1, src, dst, n), hp1, dinv,
                    b1.reshape(1, -1), g1.reshape(1, -1), be1.reshape(1, -1),
                    W2)
    hp3 = _tc_layer(_sc_message(hp2, src, dst, n), hp2, dinv,
                    b2.reshape(1, -1), g2.reshape(1, -1), be2.reshape(1, -1),
                    W3)
    out = _tc_last(_sc_message(hp3, src, dst, n), hp3, dinv,
                   b3.reshape(1, -1), g3.reshape(1, -1), be3.reshape(1, -1),
                   batch.reshape(n // BN_ROWS, 1, BN_ROWS),
                   mW1, mb1.reshape(1, -1), mW2, mb2.reshape(1, -1),
                   mW3, mb3.reshape(1, -1), oW, ob.reshape(1, -1))
    return out.reshape(G, OUT_DIM, LATENT)
